# baseline (speedup 1.0000x reference)
import functools

import jax
import jax.numpy as jnp
from jax import lax
from jax.experimental import pallas as pl
from jax.experimental.pallas import tpu as pltpu
from jax.experimental.pallas import tpu_sc as plsc

_N = 5000
_NP = 5120
_CB = 128
_NCB = _NP // _CB
_R = 128
_HS = 256
_NB = 2 * _NP
_NBE = _NB + 16
_NPE = _NP + 16


def _match_body(rx1, ry1, rx2, ry2, rid, rsc,
                cx1, cy1, cx2, cy2, cid,
                dv_out, ig_out, idx_out):
    x1 = rx1[...]
    y1 = ry1[...]
    x2 = rx2[...]
    y2 = ry2[...]
    tid = rid[...]
    aa = (x2 - x1) * (y2 - y1)
    lane = lax.broadcasted_iota(jnp.int32, (1, _CB), 1).astype(jnp.float32)

    def body(c, carry):
        nm, dm, cb = carry
        bx1 = cx1[pl.ds(c, 1), :]
        by1 = cy1[pl.ds(c, 1), :]
        bx2 = cx2[pl.ds(c, 1), :]
        by2 = cy2[pl.ds(c, 1), :]
        bid = cid[pl.ds(c, 1), :]
        iw = jnp.maximum(jnp.minimum(x2, bx2) - jnp.maximum(x1, bx1), 0.0)
        ih = jnp.maximum(jnp.minimum(y2, by2) - jnp.maximum(y1, by1), 0.0)
        inter = iw * ih
        bb = (bx2 - bx1) * (by2 - by1)
        union = aa + bb - inter
        match = tid == bid
        n = jnp.where(match, -1.0, inter)
        d = jnp.where(match, 1.0, union)
        upd = n * dm > nm * d
        nm = jnp.where(upd, n, nm)
        dm = jnp.where(upd, d, dm)
        cb = jnp.where(upd, c.astype(jnp.float32), cb)
        return nm, dm, cb

    nm0 = jnp.full((_R, _CB), -3.0, jnp.float32)
    dm0 = jnp.ones((_R, _CB), jnp.float32)
    cb0 = jnp.zeros((_R, _CB), jnp.float32)
    nm, dm, cb = lax.fori_loop(0, _NCB, body, (nm0, dm0, cb0), unroll=8)

    cm = nm / dm
    gmax = jnp.max(cm, axis=1, keepdims=True)
    jf = jnp.where(cm == gmax, cb * float(_CB) + lane, 1e9)
    bidx = jnp.min(jf, axis=1, keepdims=True)
    has = gmax > 0.0
    ig = 1.0 - jnp.where(has, gmax, 0.0)
    dv_out[...] = ig * rsc[...]
    ig_out[...] = jnp.where(has, ig, -1.0)
    idx_out[...] = bidx.astype(jnp.int32)


def _match(rx1, ry1, rx2, ry2, rid, rsc, cx1, cy1, cx2, cy2, cid,
           interpret=False):
    row_spec = pl.BlockSpec((_R, 1), lambda g: (g, 0))
    col_spec = pl.BlockSpec((_NCB, _CB), lambda g: (0, 0))
    return pl.pallas_call(
        _match_body,
        grid=(_NP // _R,),
        in_specs=[row_spec] * 6 + [col_spec] * 5,
        out_specs=[row_spec, row_spec, row_spec],
        out_shape=[
            jax.ShapeDtypeStruct((_NP, 1), jnp.float32),
            jax.ShapeDtypeStruct((_NP, 1), jnp.float32),
            jax.ShapeDtypeStruct((_NP, 1), jnp.int32),
        ],
        compiler_params=pltpu.CompilerParams(
            dimension_semantics=("parallel",)),
        interpret=interpret,
    )(rx1, ry1, rx2, ry2, rid, rsc, cx1, cy1, cx2, cy2, cid)


def _make_sc_paint():
    info = plsc.get_sparse_core_info()
    nc, ns = 1, info.num_subcores
    nw = nc * ns
    rows = _HS // nw
    strip = rows * _HS
    mesh = plsc.VectorSubcoreMesh(core_axis_name="c", subcore_axis_name="s",
                                  num_cores=1)

    @functools.partial(
        pl.kernel,
        mesh=mesh,
        out_type=jax.ShapeDtypeStruct((_HS * _HS,), jnp.float32),
        compiler_params=pltpu.CompilerParams(needs_layout_passes=False),
        scratch_types=[
            pltpu.VMEM((_NBE,), jnp.int32),
            pltpu.VMEM((_NBE,), jnp.float32),
            pltpu.VMEM((_NPE,), jnp.float32),
            pltpu.VMEM((_NPE,), jnp.int32),
            pltpu.VMEM((_NP,), jnp.float32),
            pltpu.VMEM((_NPE,), jnp.float32),
            pltpu.VMEM((_NBE,), jnp.int32),
            pltpu.VMEM((16,), jnp.int32),
            pltpu.VMEM((16,), jnp.float32),
            pltpu.VMEM((strip,), jnp.float32),
        ],
    )
    def sc_paint(pack_h, dv_h, ign_h, bidx_h, sp_h,
                 out_h,
                 packv, valsv, ignv, bidxv, spv, igmv,
                 listv, kscr, vscr, stripv):
        wid = lax.axis_index("s") * nc + lax.axis_index("c")
        y0 = wid * rows
        pltpu.sync_copy(pack_h, packv)
        pltpu.sync_copy(dv_h, valsv.at[pl.ds(0, _NP)])
        pltpu.sync_copy(ign_h, ignv)
        pltpu.sync_copy(bidx_h, bidxv)
        pltpu.sync_copy(sp_h, spv)

        iota = lax.iota(jnp.int32, 16)
        zero16 = jnp.zeros((16,), jnp.float32)
        neg16 = jnp.full((16,), -1.0, jnp.float32)

        def z_body(i, _):
            stripv[pl.ds(i * 16, 16)] = zero16
            return 0

        lax.fori_loop(0, strip // 16, z_body, 0)

        def n_body(i, _):
            igmv[pl.ds(i * 16, 16)] = neg16
            return 0

        lax.fori_loop(0, _NPE // 16, n_body, 0)

        def cand_body(g, off):
            b0 = g * 16
            pw = packv[pl.ds(b0, 16)]
            ylo = lax.shift_right_logical(pw, 8) & 255
            yhi = lax.shift_right_logical(pw, 24) & 255
            m = (ylo < y0 + rows) & (yhi > y0)
            plsc.store_compressed(listv.at[pl.ds(off, 16)], b0 + iota,
                                  mask=m)
            return off + plsc.all_reduce_population_count(m)[0]

        cnt = lax.fori_loop(0, _NB // 16, cand_body, 0)

        def igm_body(c, _):
            j16 = bidxv[pl.ds(c * 16, 16)]
            g16 = ignv[pl.ds(c * 16, 16)]
            key = jnp.where(g16 >= 0.0, j16, _NP)
            ks, vs = plsc.sort_key_val(key, g16)
            kscr[...] = ks
            for s in (1, 2, 4, 8):
                vscr[...] = vs
                idx = jnp.maximum(iota - s, 0)
                pk = plsc.load_gather(kscr, [idx])
                pv = plsc.load_gather(vscr, [idx])
                take = (iota >= s) & (pk == ks)
                vs = jnp.where(take, jnp.maximum(vs, pv), vs)
            nk = plsc.load_gather(kscr, [jnp.minimum(iota + 1, 15)])
            last = (iota == 15) | (nk != ks)
            msk = last & (ks < _NP)
            cur = plsc.load_gather(igmv, [ks])
            plsc.store_scatter(igmv, [ks], jnp.maximum(cur, vs), mask=msk)
            return 0

        lax.fori_loop(0, _NP // 16, igm_body, 0)

        def pv_body(i, _):
            g = igmv[pl.ds(i * 16, 16)]
            s = spv[pl.ds(i * 16, 16)]
            valsv[pl.ds(_NP + i * 16, 16)] = jnp.where(g >= 0.0, g * s, s)
            return 0

        lax.fori_loop(0, _NP // 16, pv_body, 0)

        def body(t, _):
            b = listv[pl.ds(t, 16)][0]
            pw = packv[pl.ds(b, 16)][0]
            x1 = pw & 255
            y1 = lax.shift_right_logical(pw, 8) & 255
            x2 = lax.shift_right_logical(pw, 16) & 255
            y2 = lax.shift_right_logical(pw, 24) & 255
            v = valsv[pl.ds(b, 16)][0]
            lo = jnp.maximum(y1, y0)
            hi = jnp.minimum(y2, y0 + rows)
            msk = (x1 + iota) < x2

            def row(y, _):
                idx = (y - y0) * _HS + x1 + iota
                cur = plsc.load_gather(stripv, [idx])
                plsc.store_scatter(stripv, [idx],
                                   jnp.maximum(cur, v), mask=msk)
                return 0

            lax.fori_loop(lo, hi, row, 0)
            return 0

        lax.fori_loop(0, cnt, body, 0)
        pltpu.sync_copy(stripv, out_h.at[pl.ds(wid * strip, strip)])

    return sc_paint


def _pad1(x):
    return jnp.pad(x, (0, _NP - x.shape[0]))


def kernel(inputs, dets, dets_prev):
    n_, c_, h, w = inputs.shape
    q = jnp.floor(dets[:, :4] * 0.5)
    qp = jnp.floor(dets_prev[:, :4] * 0.5)

    rx1 = _pad1(q[:, 0]).reshape(_NP, 1)
    ry1 = _pad1(q[:, 1]).reshape(_NP, 1)
    rx2 = _pad1(q[:, 2]).reshape(_NP, 1)
    ry2 = _pad1(q[:, 3]).reshape(_NP, 1)
    rid = _pad1(dets[:, 5]).reshape(_NP, 1)
    rsc = _pad1(dets[:, 4]).reshape(_NP, 1)
    cx1 = _pad1(qp[:, 0]).reshape(_NCB, _CB)
    cy1 = _pad1(qp[:, 1]).reshape(_NCB, _CB)
    cx2 = _pad1(qp[:, 2]).reshape(_NCB, _CB)
    cy2 = _pad1(qp[:, 3]).reshape(_NCB, _CB)
    cid = _pad1(dets_prev[:, 5]).reshape(_NCB, _CB)

    dv, ign, bidx = _match(rx1, ry1, rx2, ry2, rid, rsc,
                           cx1, cy1, cx2, cy2, cid)
    dv = dv.reshape(_NP)
    ign = ign.reshape(_NP)
    bidx = bidx.reshape(_NP)

    qi = q.astype(jnp.int32)
    qpi = qp.astype(jnp.int32)

    def packed(qb):
        return (qb[:, 0] | (qb[:, 1] << 8) | (qb[:, 2] << 16)
                | (qb[:, 3] << 24))

    pack = jnp.pad(jnp.concatenate([_pad1(packed(qi)), _pad1(packed(qpi))]),
                   (0, 16))
    sp = _pad1(dets_prev[:, 4])
    ign_e = jnp.pad(ign, (0, 16))
    bidx_e = jnp.pad(bidx, (0, 16))

    m_flat = _make_sc_paint()(pack, dv, ign_e, bidx_e, sp)
    m = m_flat.reshape(_HS, _HS)
    up = jnp.broadcast_to(m[:, None, :, None], (_HS, 2, _HS, 2))
    return up.reshape(1, 1, h, w)


# speedup vs baseline: 1.0929x; 1.0929x over previous
import functools

import jax
import jax.numpy as jnp
from jax import lax
from jax.experimental import pallas as pl
from jax.experimental.pallas import tpu as pltpu
from jax.experimental.pallas import tpu_sc as plsc

_N = 5000
_NP = 5120
_CB = 128
_NCB = _NP // _CB
_R = 128
_HS = 256
_NB = 2 * _NP
_NBE = _NB + 16
_NPE = _NP + 16


def _match_body(rx1, ry1, rx2, ry2, rid, rsc,
                cx1, cy1, cx2, cy2, cid,
                dv_out, ig_out, idx_out):
    x1 = rx1[...]
    y1 = ry1[...]
    x2 = rx2[...]
    y2 = ry2[...]
    tid = rid[...]
    aa = (x2 - x1) * (y2 - y1)
    lane = lax.broadcasted_iota(jnp.int32, (1, _CB), 1).astype(jnp.float32)

    def body(c, carry):
        nm, dm, cb = carry
        bx1 = cx1[pl.ds(c, 1), :]
        by1 = cy1[pl.ds(c, 1), :]
        bx2 = cx2[pl.ds(c, 1), :]
        by2 = cy2[pl.ds(c, 1), :]
        bid = cid[pl.ds(c, 1), :]
        iw = jnp.maximum(jnp.minimum(x2, bx2) - jnp.maximum(x1, bx1), 0.0)
        ih = jnp.maximum(jnp.minimum(y2, by2) - jnp.maximum(y1, by1), 0.0)
        inter = iw * ih
        bb = (bx2 - bx1) * (by2 - by1)
        union = aa + bb - inter
        match = tid == bid
        n = jnp.where(match, -1.0, inter)
        d = jnp.where(match, 1.0, union)
        upd = n * dm > nm * d
        nm = jnp.where(upd, n, nm)
        dm = jnp.where(upd, d, dm)
        cb = jnp.where(upd, c.astype(jnp.float32), cb)
        return nm, dm, cb

    nm0 = jnp.full((_R, _CB), -3.0, jnp.float32)
    dm0 = jnp.ones((_R, _CB), jnp.float32)
    cb0 = jnp.zeros((_R, _CB), jnp.float32)
    nm, dm, cb = lax.fori_loop(0, _NCB, body, (nm0, dm0, cb0), unroll=8)

    cm = nm / dm
    gmax = jnp.max(cm, axis=1, keepdims=True)
    jf = jnp.where(cm == gmax, cb * float(_CB) + lane, 1e9)
    bidx = jnp.min(jf, axis=1, keepdims=True)
    has = gmax > 0.0
    ig = 1.0 - jnp.where(has, gmax, 0.0)
    dv_out[...] = ig * rsc[...]
    ig_out[...] = jnp.where(has, ig, -1.0)
    idx_out[...] = bidx.astype(jnp.int32)


def _match(rx1, ry1, rx2, ry2, rid, rsc, cx1, cy1, cx2, cy2, cid,
           interpret=False):
    row_spec = pl.BlockSpec((_R, 1), lambda g: (g, 0))
    col_spec = pl.BlockSpec((_NCB, _CB), lambda g: (0, 0))
    return pl.pallas_call(
        _match_body,
        grid=(_NP // _R,),
        in_specs=[row_spec] * 6 + [col_spec] * 5,
        out_specs=[row_spec, row_spec, row_spec],
        out_shape=[
            jax.ShapeDtypeStruct((_NP, 1), jnp.float32),
            jax.ShapeDtypeStruct((_NP, 1), jnp.float32),
            jax.ShapeDtypeStruct((_NP, 1), jnp.int32),
        ],
        compiler_params=pltpu.CompilerParams(
            dimension_semantics=("parallel",)),
        interpret=interpret,
    )(rx1, ry1, rx2, ry2, rid, rsc, cx1, cy1, cx2, cy2, cid)


def _make_sc_paint():
    info = plsc.get_sparse_core_info()
    nc, ns = info.num_cores, info.num_subcores
    nw = nc * ns
    rows = _HS // nw
    strip = rows * _HS
    mesh = plsc.VectorSubcoreMesh(core_axis_name="c", subcore_axis_name="s")

    @functools.partial(
        pl.kernel,
        mesh=mesh,
        out_type=jax.ShapeDtypeStruct((_HS * _HS,), jnp.float32),
        compiler_params=pltpu.CompilerParams(needs_layout_passes=False),
        scratch_types=[
            pltpu.VMEM((_NBE,), jnp.int32),
            pltpu.VMEM((_NBE,), jnp.float32),
            pltpu.VMEM((_NPE,), jnp.float32),
            pltpu.VMEM((_NPE,), jnp.int32),
            pltpu.VMEM((_NP,), jnp.float32),
            pltpu.VMEM((_NPE,), jnp.float32),
            pltpu.VMEM((_NBE,), jnp.int32),
            pltpu.VMEM((16,), jnp.int32),
            pltpu.VMEM((16,), jnp.float32),
            pltpu.VMEM((strip,), jnp.float32),
        ],
    )
    def sc_paint(pack_h, dv_h, ign_h, bidx_h, sp_h,
                 out_h,
                 packv, valsv, ignv, bidxv, spv, igmv,
                 listv, kscr, vscr, stripv):
        wid = lax.axis_index("s") * nc + lax.axis_index("c")
        y0 = wid * rows
        pltpu.sync_copy(pack_h, packv)
        pltpu.sync_copy(dv_h, valsv.at[pl.ds(0, _NP)])
        pltpu.sync_copy(ign_h, ignv)
        pltpu.sync_copy(bidx_h, bidxv)
        pltpu.sync_copy(sp_h, spv)

        iota = lax.iota(jnp.int32, 16)
        zero16 = jnp.zeros((16,), jnp.float32)
        neg16 = jnp.full((16,), -1.0, jnp.float32)

        def z_body(i, _):
            stripv[pl.ds(i * 16, 16)] = zero16
            return 0

        lax.fori_loop(0, strip // 16, z_body, 0)

        def n_body(i, _):
            igmv[pl.ds(i * 16, 16)] = neg16
            return 0

        lax.fori_loop(0, _NPE // 16, n_body, 0)

        def cand_body(g, off):
            b0 = g * 16
            pw = packv[pl.ds(b0, 16)]
            ylo = lax.shift_right_logical(pw, 8) & 255
            yhi = lax.shift_right_logical(pw, 24) & 255
            m = (ylo < y0 + rows) & (yhi > y0)
            plsc.store_compressed(listv.at[pl.ds(off, 16)], b0 + iota,
                                  mask=m)
            return off + plsc.all_reduce_population_count(m)[0]

        cnt = lax.fori_loop(0, _NB // 16, cand_body, 0)

        def igm_body(c, _):
            j16 = bidxv[pl.ds(c * 16, 16)]
            g16 = ignv[pl.ds(c * 16, 16)]
            key = jnp.where(g16 >= 0.0, j16, _NP)
            ks, vs = plsc.sort_key_val(key, g16)
            kscr[...] = ks
            for s in (1, 2, 4, 8):
                vscr[...] = vs
                idx = jnp.maximum(iota - s, 0)
                pk = plsc.load_gather(kscr, [idx])
                pv = plsc.load_gather(vscr, [idx])
                take = (iota >= s) & (pk == ks)
                vs = jnp.where(take, jnp.maximum(vs, pv), vs)
            nk = plsc.load_gather(kscr, [jnp.minimum(iota + 1, 15)])
            last = (iota == 15) | (nk != ks)
            msk = last & (ks < _NP)
            cur = plsc.load_gather(igmv, [ks])
            plsc.store_scatter(igmv, [ks], jnp.maximum(cur, vs), mask=msk)
            return 0

        lax.fori_loop(0, _NP // 16, igm_body, 0)

        def pv_body(i, _):
            g = igmv[pl.ds(i * 16, 16)]
            s = spv[pl.ds(i * 16, 16)]
            valsv[pl.ds(_NP + i * 16, 16)] = jnp.where(g >= 0.0, g * s, s)
            return 0

        lax.fori_loop(0, _NP // 16, pv_body, 0)

        def body(t, _):
            b = listv[pl.ds(t, 16)][0]
            pw = packv[pl.ds(b, 16)][0]
            x1 = pw & 255
            y1 = lax.shift_right_logical(pw, 8) & 255
            x2 = lax.shift_right_logical(pw, 16) & 255
            y2 = lax.shift_right_logical(pw, 24) & 255
            v = valsv[pl.ds(b, 16)][0]
            lo = jnp.maximum(y1, y0)
            hi = jnp.minimum(y2, y0 + rows)
            msk = (x1 + iota) < x2

            def row(y, _):
                idx = (y - y0) * _HS + x1 + iota
                cur = plsc.load_gather(stripv, [idx])
                plsc.store_scatter(stripv, [idx],
                                   jnp.maximum(cur, v), mask=msk)
                return 0

            lax.fori_loop(lo, hi, row, 0)
            return 0

        lax.fori_loop(0, cnt, body, 0)
        pltpu.sync_copy(stripv, out_h.at[pl.ds(wid * strip, strip)])

    return sc_paint


def _pad1(x):
    return jnp.pad(x, (0, _NP - x.shape[0]))


def kernel(inputs, dets, dets_prev):
    n_, c_, h, w = inputs.shape
    q = jnp.floor(dets[:, :4] * 0.5)
    qp = jnp.floor(dets_prev[:, :4] * 0.5)

    rx1 = _pad1(q[:, 0]).reshape(_NP, 1)
    ry1 = _pad1(q[:, 1]).reshape(_NP, 1)
    rx2 = _pad1(q[:, 2]).reshape(_NP, 1)
    ry2 = _pad1(q[:, 3]).reshape(_NP, 1)
    rid = _pad1(dets[:, 5]).reshape(_NP, 1)
    rsc = _pad1(dets[:, 4]).reshape(_NP, 1)
    cx1 = _pad1(qp[:, 0]).reshape(_NCB, _CB)
    cy1 = _pad1(qp[:, 1]).reshape(_NCB, _CB)
    cx2 = _pad1(qp[:, 2]).reshape(_NCB, _CB)
    cy2 = _pad1(qp[:, 3]).reshape(_NCB, _CB)
    cid = _pad1(dets_prev[:, 5]).reshape(_NCB, _CB)

    dv, ign, bidx = _match(rx1, ry1, rx2, ry2, rid, rsc,
                           cx1, cy1, cx2, cy2, cid)
    dv = dv.reshape(_NP)
    ign = ign.reshape(_NP)
    bidx = bidx.reshape(_NP)

    qi = q.astype(jnp.int32)
    qpi = qp.astype(jnp.int32)

    def packed(qb):
        return (qb[:, 0] | (qb[:, 1] << 8) | (qb[:, 2] << 16)
                | (qb[:, 3] << 24))

    pack = jnp.pad(jnp.concatenate([_pad1(packed(qi)), _pad1(packed(qpi))]),
                   (0, 16))
    sp = _pad1(dets_prev[:, 4])
    ign_e = jnp.pad(ign, (0, 16))
    bidx_e = jnp.pad(bidx, (0, 16))

    m_flat = _make_sc_paint()(pack, dv, ign_e, bidx_e, sp)
    m = m_flat.reshape(_HS, _HS)
    up = jnp.broadcast_to(m[:, None, :, None], (_HS, 2, _HS, 2))
    return up.reshape(1, 1, h, w)


# speedup vs baseline: 1.1274x; 1.0316x over previous
import functools

import jax
import jax.numpy as jnp
from jax import lax
from jax.experimental import pallas as pl
from jax.experimental.pallas import tpu as pltpu
from jax.experimental.pallas import tpu_sc as plsc

_N = 5000
_NP = 5120
_CB = 128
_NCB = _NP // _CB
_R = 128
_HS = 256
_NB = 2 * _NP
_NBE = _NB + 16
_NPE = _NP + 16


def _match_body(rx1, ry1, rx2, ry2, rid, rsc,
                cx1, cy1, cx2, cy2, cid,
                dv_out, ig_out, idx_out):
    x1 = rx1[...]
    y1 = ry1[...]
    x2 = rx2[...]
    y2 = ry2[...]
    tid = rid[...]
    aa = (x2 - x1) * (y2 - y1)
    lane = lax.broadcasted_iota(jnp.int32, (1, _CB), 1).astype(jnp.float32)

    def body(c, carry):
        nm, dm, cb = carry
        bx1 = cx1[pl.ds(c, 1), :]
        by1 = cy1[pl.ds(c, 1), :]
        bx2 = cx2[pl.ds(c, 1), :]
        by2 = cy2[pl.ds(c, 1), :]
        bid = cid[pl.ds(c, 1), :]
        iw = jnp.maximum(jnp.minimum(x2, bx2) - jnp.maximum(x1, bx1), 0.0)
        ih = jnp.maximum(jnp.minimum(y2, by2) - jnp.maximum(y1, by1), 0.0)
        inter = iw * ih
        bb = (bx2 - bx1) * (by2 - by1)
        union = aa + bb - inter
        match = tid == bid
        n = jnp.where(match, -1.0, inter)
        d = jnp.where(match, 1.0, union)
        upd = n * dm > nm * d
        nm = jnp.where(upd, n, nm)
        dm = jnp.where(upd, d, dm)
        cb = jnp.where(upd, c.astype(jnp.float32), cb)
        return nm, dm, cb

    nm0 = jnp.full((_R, _CB), -3.0, jnp.float32)
    dm0 = jnp.ones((_R, _CB), jnp.float32)
    cb0 = jnp.zeros((_R, _CB), jnp.float32)
    nm, dm, cb = lax.fori_loop(0, _NCB, body, (nm0, dm0, cb0), unroll=20)

    cm = nm / dm
    gmax = jnp.max(cm, axis=1, keepdims=True)
    jf = jnp.where(cm == gmax, cb * float(_CB) + lane, 1e9)
    bidx = jnp.min(jf, axis=1, keepdims=True)
    has = gmax > 0.0
    ig = 1.0 - jnp.where(has, gmax, 0.0)
    dv_out[...] = ig * rsc[...]
    ig_out[...] = jnp.where(has, ig, -1.0)
    idx_out[...] = bidx.astype(jnp.int32)


def _match(rx1, ry1, rx2, ry2, rid, rsc, cx1, cy1, cx2, cy2, cid,
           interpret=False):
    row_spec = pl.BlockSpec((_R, 1), lambda g: (g, 0))
    col_spec = pl.BlockSpec((_NCB, _CB), lambda g: (0, 0))
    return pl.pallas_call(
        _match_body,
        grid=(_NP // _R,),
        in_specs=[row_spec] * 6 + [col_spec] * 5,
        out_specs=[row_spec, row_spec, row_spec],
        out_shape=[
            jax.ShapeDtypeStruct((_NP, 1), jnp.float32),
            jax.ShapeDtypeStruct((_NP, 1), jnp.float32),
            jax.ShapeDtypeStruct((_NP, 1), jnp.int32),
        ],
        compiler_params=pltpu.CompilerParams(
            dimension_semantics=("parallel",)),
        interpret=interpret,
    )(rx1, ry1, rx2, ry2, rid, rsc, cx1, cy1, cx2, cy2, cid)


def _make_sc_paint():
    info = plsc.get_sparse_core_info()
    nc, ns = info.num_cores, info.num_subcores
    nw = nc * ns
    rows = _HS // nw
    strip = rows * _HS
    mesh = plsc.VectorSubcoreMesh(core_axis_name="c", subcore_axis_name="s")

    @functools.partial(
        pl.kernel,
        mesh=mesh,
        out_type=jax.ShapeDtypeStruct((_HS * _HS,), jnp.float32),
        compiler_params=pltpu.CompilerParams(needs_layout_passes=False),
        scratch_types=[
            pltpu.VMEM((_NBE,), jnp.int32),
            pltpu.VMEM((_NBE,), jnp.float32),
            pltpu.VMEM((_NPE,), jnp.float32),
            pltpu.VMEM((_NPE,), jnp.int32),
            pltpu.VMEM((_NP,), jnp.float32),
            pltpu.VMEM((_NPE,), jnp.float32),
            pltpu.VMEM((_NBE,), jnp.int32),
            pltpu.VMEM((16,), jnp.int32),
            pltpu.VMEM((16,), jnp.float32),
            pltpu.VMEM((strip,), jnp.float32),
        ],
    )
    def sc_paint(pack_h, dv_h, ign_h, bidx_h, sp_h,
                 out_h,
                 packv, valsv, ignv, bidxv, spv, igmv,
                 listv, kscr, vscr, stripv):
        wid = lax.axis_index("s") * nc + lax.axis_index("c")
        y0 = wid * rows
        pltpu.sync_copy(pack_h, packv)
        pltpu.sync_copy(dv_h, valsv.at[pl.ds(0, _NP)])
        pltpu.sync_copy(ign_h, ignv)
        pltpu.sync_copy(bidx_h, bidxv)
        pltpu.sync_copy(sp_h, spv)

        iota = lax.iota(jnp.int32, 16)
        zero16 = jnp.zeros((16,), jnp.float32)
        neg16 = jnp.full((16,), -1.0, jnp.float32)

        def z_body(i, _):
            stripv[pl.ds(i * 16, 16)] = zero16
            return 0

        lax.fori_loop(0, strip // 16, z_body, 0)

        def n_body(i, _):
            igmv[pl.ds(i * 16, 16)] = neg16
            return 0

        lax.fori_loop(0, _NPE // 16, n_body, 0)

        def cand_body(g, off):
            b0 = g * 16
            pw = packv[pl.ds(b0, 16)]
            ylo = lax.shift_right_logical(pw, 8) & 255
            yhi = lax.shift_right_logical(pw, 24) & 255
            m = (ylo < y0 + rows) & (yhi > y0)
            plsc.store_compressed(listv.at[pl.ds(off, 16)], b0 + iota,
                                  mask=m)
            return off + plsc.all_reduce_population_count(m)[0]

        cnt = lax.fori_loop(0, _NB // 16, cand_body, 0)

        def igm_body(c, _):
            j16 = bidxv[pl.ds(c * 16, 16)]
            g16 = ignv[pl.ds(c * 16, 16)]
            key = jnp.where(g16 >= 0.0, j16, _NP)
            ks, vs = plsc.sort_key_val(key, g16)
            kscr[...] = ks
            for s in (1, 2, 4, 8):
                vscr[...] = vs
                idx = jnp.maximum(iota - s, 0)
                pk = plsc.load_gather(kscr, [idx])
                pv = plsc.load_gather(vscr, [idx])
                take = (iota >= s) & (pk == ks)
                vs = jnp.where(take, jnp.maximum(vs, pv), vs)
            nk = plsc.load_gather(kscr, [jnp.minimum(iota + 1, 15)])
            last = (iota == 15) | (nk != ks)
            msk = last & (ks < _NP)
            cur = plsc.load_gather(igmv, [ks])
            plsc.store_scatter(igmv, [ks], jnp.maximum(cur, vs), mask=msk)
            return 0

        lax.fori_loop(0, _NP // 16, igm_body, 0)

        def pv_body(i, _):
            g = igmv[pl.ds(i * 16, 16)]
            s = spv[pl.ds(i * 16, 16)]
            valsv[pl.ds(_NP + i * 16, 16)] = jnp.where(g >= 0.0, g * s, s)
            return 0

        lax.fori_loop(0, _NP // 16, pv_body, 0)

        def body(t, _):
            b = listv[pl.ds(t, 16)][0]
            pw = packv[pl.ds(b, 16)][0]
            x1 = pw & 255
            y1 = lax.shift_right_logical(pw, 8) & 255
            x2 = lax.shift_right_logical(pw, 16) & 255
            y2 = lax.shift_right_logical(pw, 24) & 255
            v = valsv[pl.ds(b, 16)][0]
            lo = jnp.maximum(y1, y0)
            hi = jnp.minimum(y2, y0 + rows)
            msk = (x1 + iota) < x2

            def row(y, _):
                idx = (y - y0) * _HS + x1 + iota
                cur = plsc.load_gather(stripv, [idx])
                plsc.store_scatter(stripv, [idx],
                                   jnp.maximum(cur, v), mask=msk)
                return 0

            lax.fori_loop(lo, hi, row, 0)
            return 0

        lax.fori_loop(0, cnt, body, 0)
        pltpu.sync_copy(stripv, out_h.at[pl.ds(wid * strip, strip)])

    return sc_paint


def _pad1(x):
    return jnp.pad(x, (0, _NP - x.shape[0]))


def kernel(inputs, dets, dets_prev):
    n_, c_, h, w = inputs.shape
    q = jnp.floor(dets[:, :4] * 0.5)
    qp = jnp.floor(dets_prev[:, :4] * 0.5)

    rx1 = _pad1(q[:, 0]).reshape(_NP, 1)
    ry1 = _pad1(q[:, 1]).reshape(_NP, 1)
    rx2 = _pad1(q[:, 2]).reshape(_NP, 1)
    ry2 = _pad1(q[:, 3]).reshape(_NP, 1)
    rid = _pad1(dets[:, 5]).reshape(_NP, 1)
    rsc = _pad1(dets[:, 4]).reshape(_NP, 1)
    cx1 = _pad1(qp[:, 0]).reshape(_NCB, _CB)
    cy1 = _pad1(qp[:, 1]).reshape(_NCB, _CB)
    cx2 = _pad1(qp[:, 2]).reshape(_NCB, _CB)
    cy2 = _pad1(qp[:, 3]).reshape(_NCB, _CB)
    cid = _pad1(dets_prev[:, 5]).reshape(_NCB, _CB)

    dv, ign, bidx = _match(rx1, ry1, rx2, ry2, rid, rsc,
                           cx1, cy1, cx2, cy2, cid)
    dv = dv.reshape(_NP)
    ign = ign.reshape(_NP)
    bidx = bidx.reshape(_NP)

    qi = q.astype(jnp.int32)
    qpi = qp.astype(jnp.int32)

    def packed(qb):
        return (qb[:, 0] | (qb[:, 1] << 8) | (qb[:, 2] << 16)
                | (qb[:, 3] << 24))

    pack = jnp.pad(jnp.concatenate([_pad1(packed(qi)), _pad1(packed(qpi))]),
                   (0, 16))
    sp = _pad1(dets_prev[:, 4])
    ign_e = jnp.pad(ign, (0, 16))
    bidx_e = jnp.pad(bidx, (0, 16))

    m_flat = _make_sc_paint()(pack, dv, ign_e, bidx_e, sp)
    m = m_flat.reshape(_HS, _HS)
    up = jnp.broadcast_to(m[:, None, :, None], (_HS, 2, _HS, 2))
    return up.reshape(1, 1, h, w)


# speedup vs baseline: 1.4622x; 1.2970x over previous
import functools

import jax
import jax.numpy as jnp
from jax import lax
from jax.experimental import pallas as pl
from jax.experimental.pallas import tpu as pltpu
from jax.experimental.pallas import tpu_sc as plsc

_N = 5000
_NP = 5120
_CB = 128
_NCB = _NP // _CB
_R = 128
_HS = 256
_NB = 2 * _NP
_NBE = _NB + 16
_NPE = _NP + 16


def _match_body(rx1, ry1, rx2, ry2, rid, rsc,
                cx1, cy1, cx2, cy2, cid,
                dv_out, ig_out, idx_out):
    x1 = rx1[...]
    y1 = ry1[...]
    x2 = rx2[...]
    y2 = ry2[...]
    tid = rid[...]
    aa = (x2 - x1) * (y2 - y1)
    lane = lax.broadcasted_iota(jnp.int32, (1, _CB), 1).astype(jnp.float32)

    def body(c, carry):
        nm, dm, cb = carry
        bx1 = cx1[pl.ds(c, 1), :]
        by1 = cy1[pl.ds(c, 1), :]
        bx2 = cx2[pl.ds(c, 1), :]
        by2 = cy2[pl.ds(c, 1), :]
        bid = cid[pl.ds(c, 1), :]
        iw = jnp.maximum(jnp.minimum(x2, bx2) - jnp.maximum(x1, bx1), 0.0)
        ih = jnp.maximum(jnp.minimum(y2, by2) - jnp.maximum(y1, by1), 0.0)
        inter = iw * ih
        bb = (bx2 - bx1) * (by2 - by1)
        union = aa + bb - inter
        match = tid == bid
        n = jnp.where(match, -1.0, inter)
        d = jnp.where(match, 1.0, union)
        upd = n * dm > nm * d
        nm = jnp.where(upd, n, nm)
        dm = jnp.where(upd, d, dm)
        cb = jnp.where(upd, c.astype(jnp.float32), cb)
        return nm, dm, cb

    nm0 = jnp.full((_R, _CB), -3.0, jnp.float32)
    dm0 = jnp.ones((_R, _CB), jnp.float32)
    cb0 = jnp.zeros((_R, _CB), jnp.float32)
    nm, dm, cb = lax.fori_loop(0, _NCB, body, (nm0, dm0, cb0), unroll=20)

    cm = nm / dm
    gmax = jnp.max(cm, axis=1, keepdims=True)
    jf = jnp.where(cm == gmax, cb * float(_CB) + lane, 1e9)
    bidx = jnp.min(jf, axis=1, keepdims=True)
    has = gmax > 0.0
    ig = 1.0 - jnp.where(has, gmax, 0.0)
    dv_out[...] = ig * rsc[...]
    ig_out[...] = jnp.where(has, ig, -1.0)
    idx_out[...] = bidx.astype(jnp.int32)


def _match(rx1, ry1, rx2, ry2, rid, rsc, cx1, cy1, cx2, cy2, cid,
           interpret=False):
    row_spec = pl.BlockSpec((_R, 1), lambda g: (g, 0))
    col_spec = pl.BlockSpec((_NCB, _CB), lambda g: (0, 0))
    return pl.pallas_call(
        _match_body,
        grid=(_NP // _R,),
        in_specs=[row_spec] * 6 + [col_spec] * 5,
        out_specs=[row_spec, row_spec, row_spec],
        out_shape=[
            jax.ShapeDtypeStruct((_NP, 1), jnp.float32),
            jax.ShapeDtypeStruct((_NP, 1), jnp.float32),
            jax.ShapeDtypeStruct((_NP, 1), jnp.int32),
        ],
        compiler_params=pltpu.CompilerParams(
            dimension_semantics=("parallel",)),
        interpret=interpret,
    )(rx1, ry1, rx2, ry2, rid, rsc, cx1, cy1, cx2, cy2, cid)


def _make_sc_paint():
    info = plsc.get_sparse_core_info()
    nc, ns = info.num_cores, info.num_subcores
    nw = nc * ns
    rows = _HS // nw
    strip = rows * _HS
    mesh = plsc.VectorSubcoreMesh(core_axis_name="c", subcore_axis_name="s")

    @functools.partial(
        pl.kernel,
        mesh=mesh,
        out_type=jax.ShapeDtypeStruct((4 * _HS * _HS,), jnp.float32),
        compiler_params=pltpu.CompilerParams(needs_layout_passes=False),
        scratch_types=[
            pltpu.VMEM((_NBE,), jnp.int32),
            pltpu.VMEM((_NBE,), jnp.float32),
            pltpu.VMEM((_NPE,), jnp.float32),
            pltpu.VMEM((_NPE,), jnp.int32),
            pltpu.VMEM((_NP,), jnp.float32),
            pltpu.VMEM((_NPE,), jnp.float32),
            pltpu.VMEM((_NBE,), jnp.int32),
            pltpu.VMEM((16,), jnp.int32),
            pltpu.VMEM((16,), jnp.float32),
            pltpu.VMEM((strip,), jnp.float32),
            pltpu.VMEM((4 * strip,), jnp.float32),
        ],
    )
    def sc_paint(pack_h, dv_h, ign_h, bidx_h, sp_h,
                 out_h,
                 packv, valsv, ignv, bidxv, spv, igmv,
                 listv, kscr, vscr, stripv, upv):
        wid = lax.axis_index("s") * nc + lax.axis_index("c")
        y0 = wid * rows
        pltpu.sync_copy(pack_h, packv)
        pltpu.sync_copy(dv_h, valsv.at[pl.ds(0, _NP)])
        pltpu.sync_copy(ign_h, ignv)
        pltpu.sync_copy(bidx_h, bidxv)
        pltpu.sync_copy(sp_h, spv)

        iota = lax.iota(jnp.int32, 16)
        zero16 = jnp.zeros((16,), jnp.float32)
        neg16 = jnp.full((16,), -1.0, jnp.float32)

        def z_body(i, _):
            stripv[pl.ds(i * 16, 16)] = zero16
            return 0

        lax.fori_loop(0, strip // 16, z_body, 0)

        def n_body(i, _):
            igmv[pl.ds(i * 16, 16)] = neg16
            return 0

        lax.fori_loop(0, _NPE // 16, n_body, 0)

        def cand_body(g, off):
            b0 = g * 16
            pw = packv[pl.ds(b0, 16)]
            ylo = lax.shift_right_logical(pw, 8) & 255
            yhi = lax.shift_right_logical(pw, 24) & 255
            m = (ylo < y0 + rows) & (yhi > y0)
            plsc.store_compressed(listv.at[pl.ds(off, 16)], b0 + iota,
                                  mask=m)
            return off + plsc.all_reduce_population_count(m)[0]

        cnt = lax.fori_loop(0, _NB // 16, cand_body, 0)

        def igm_body(c, _):
            j16 = bidxv[pl.ds(c * 16, 16)]
            g16 = ignv[pl.ds(c * 16, 16)]
            key = jnp.where(g16 >= 0.0, j16, _NP)
            ks, vs = plsc.sort_key_val(key, g16)
            kscr[...] = ks
            for s in (1, 2, 4, 8):
                vscr[...] = vs
                idx = jnp.maximum(iota - s, 0)
                pk = plsc.load_gather(kscr, [idx])
                pv = plsc.load_gather(vscr, [idx])
                take = (iota >= s) & (pk == ks)
                vs = jnp.where(take, jnp.maximum(vs, pv), vs)
            nk = plsc.load_gather(kscr, [jnp.minimum(iota + 1, 15)])
            last = (iota == 15) | (nk != ks)
            msk = last & (ks < _NP)
            cur = plsc.load_gather(igmv, [ks])
            plsc.store_scatter(igmv, [ks], jnp.maximum(cur, vs), mask=msk)
            return 0

        lax.fori_loop(0, _NP // 16, igm_body, 0)

        def pv_body(i, _):
            g = igmv[pl.ds(i * 16, 16)]
            s = spv[pl.ds(i * 16, 16)]
            valsv[pl.ds(_NP + i * 16, 16)] = jnp.where(g >= 0.0, g * s, s)
            return 0

        lax.fori_loop(0, _NP // 16, pv_body, 0)

        def body(t, _):
            b = listv[pl.ds(t, 16)][0]
            pw = packv[pl.ds(b, 16)][0]
            x1 = pw & 255
            y1 = lax.shift_right_logical(pw, 8) & 255
            x2 = lax.shift_right_logical(pw, 16) & 255
            y2 = lax.shift_right_logical(pw, 24) & 255
            v = valsv[pl.ds(b, 16)][0]
            lo = jnp.maximum(y1, y0)
            hi = jnp.minimum(y2, y0 + rows)
            msk = (x1 + iota) < x2

            def row(y, _):
                idx = (y - y0) * _HS + x1 + iota
                cur = plsc.load_gather(stripv, [idx])
                plsc.store_scatter(stripv, [idx],
                                   jnp.maximum(cur, v), mask=msk)
                return 0

            lax.fori_loop(lo, hi, row, 0)
            return 0

        lax.fori_loop(0, cnt, body, 0)

        def up_body(r, _):
            def chunk(k, _):
                src = plsc.load_gather(
                    stripv,
                    [r * _HS + lax.shift_right_logical(k * 16 + iota, 1)])
                upv[pl.ds(r * 4 * _HS + k * 16, 16)] = src
                upv[pl.ds(r * 4 * _HS + 2 * _HS + k * 16, 16)] = src
                return 0

            lax.fori_loop(0, 2 * _HS // 16, chunk, 0)
            return 0

        lax.fori_loop(0, rows, up_body, 0)
        pltpu.sync_copy(upv, out_h.at[pl.ds(wid * 4 * strip, 4 * strip)])

    return sc_paint


def _pad1(x):
    return jnp.pad(x, (0, _NP - x.shape[0]))


def kernel(inputs, dets, dets_prev):
    n_, c_, h, w = inputs.shape
    q = jnp.floor(dets[:, :4] * 0.5)
    qp = jnp.floor(dets_prev[:, :4] * 0.5)

    rx1 = _pad1(q[:, 0]).reshape(_NP, 1)
    ry1 = _pad1(q[:, 1]).reshape(_NP, 1)
    rx2 = _pad1(q[:, 2]).reshape(_NP, 1)
    ry2 = _pad1(q[:, 3]).reshape(_NP, 1)
    rid = _pad1(dets[:, 5]).reshape(_NP, 1)
    rsc = _pad1(dets[:, 4]).reshape(_NP, 1)
    cx1 = _pad1(qp[:, 0]).reshape(_NCB, _CB)
    cy1 = _pad1(qp[:, 1]).reshape(_NCB, _CB)
    cx2 = _pad1(qp[:, 2]).reshape(_NCB, _CB)
    cy2 = _pad1(qp[:, 3]).reshape(_NCB, _CB)
    cid = _pad1(dets_prev[:, 5]).reshape(_NCB, _CB)

    dv, ign, bidx = _match(rx1, ry1, rx2, ry2, rid, rsc,
                           cx1, cy1, cx2, cy2, cid)
    dv = dv.reshape(_NP)
    ign = ign.reshape(_NP)
    bidx = bidx.reshape(_NP)

    qi = q.astype(jnp.int32)
    qpi = qp.astype(jnp.int32)

    def packed(qb):
        return (qb[:, 0] | (qb[:, 1] << 8) | (qb[:, 2] << 16)
                | (qb[:, 3] << 24))

    pack = jnp.pad(jnp.concatenate([_pad1(packed(qi)), _pad1(packed(qpi))]),
                   (0, 16))
    sp = _pad1(dets_prev[:, 4])
    ign_e = jnp.pad(ign, (0, 16))
    bidx_e = jnp.pad(bidx, (0, 16))

    m_flat = _make_sc_paint()(pack, dv, ign_e, bidx_e, sp)
    return m_flat.reshape(1, 1, h, w)


# speedup vs baseline: 1.5444x; 1.0562x over previous
import functools

import jax
import jax.numpy as jnp
from jax import lax
from jax.experimental import pallas as pl
from jax.experimental.pallas import tpu as pltpu
from jax.experimental.pallas import tpu_sc as plsc

_N = 5000
_NP = 5120
_CB = 128
_NCB = _NP // _CB
_R = 128
_HS = 256
_NB = 2 * _NP
_NBE = _NB + 16
_NPE = _NP + 16


def _match_body(rx1, ry1, rx2, ry2, rid, rsc,
                cx1, cy1, cx2, cy2, cid,
                dv_out, ig_out, idx_out):
    x1 = rx1[...].reshape(_R, 1)
    y1 = ry1[...].reshape(_R, 1)
    x2 = rx2[...].reshape(_R, 1)
    y2 = ry2[...].reshape(_R, 1)
    tid = rid[...].reshape(_R, 1)
    aa = (x2 - x1) * (y2 - y1)
    lane = lax.broadcasted_iota(jnp.int32, (1, _CB), 1).astype(jnp.float32)

    def body(c, carry):
        nm, dm, cb = carry
        bx1 = cx1[pl.ds(c, 1), :]
        by1 = cy1[pl.ds(c, 1), :]
        bx2 = cx2[pl.ds(c, 1), :]
        by2 = cy2[pl.ds(c, 1), :]
        bid = cid[pl.ds(c, 1), :]
        iw = jnp.maximum(jnp.minimum(x2, bx2) - jnp.maximum(x1, bx1), 0.0)
        ih = jnp.maximum(jnp.minimum(y2, by2) - jnp.maximum(y1, by1), 0.0)
        inter = iw * ih
        bb = (bx2 - bx1) * (by2 - by1)
        union = aa + bb - inter
        match = tid == bid
        n = jnp.where(match, -1.0, inter)
        d = jnp.where(match, 1.0, union)
        upd = n * dm > nm * d
        nm = jnp.where(upd, n, nm)
        dm = jnp.where(upd, d, dm)
        cb = jnp.where(upd, c.astype(jnp.float32), cb)
        return nm, dm, cb

    nm0 = jnp.full((_R, _CB), -3.0, jnp.float32)
    dm0 = jnp.ones((_R, _CB), jnp.float32)
    cb0 = jnp.zeros((_R, _CB), jnp.float32)
    nm, dm, cb = lax.fori_loop(0, _NCB, body, (nm0, dm0, cb0), unroll=20)

    cm = nm / dm
    gmax = jnp.max(cm, axis=1, keepdims=True)
    jf = jnp.where(cm == gmax, cb * float(_CB) + lane, 1e9)
    bidx = jnp.min(jf, axis=1, keepdims=True)
    has = gmax > 0.0
    ig = 1.0 - jnp.where(has, gmax, 0.0)
    dv_out[...] = (ig * rsc[...].reshape(_R, 1)).reshape(1, 1, _R)
    ig_out[...] = jnp.where(has, ig, -1.0).reshape(1, 1, _R)
    idx_out[...] = bidx.astype(jnp.int32).reshape(1, 1, _R)


def _match(rx1, ry1, rx2, ry2, rid, rsc, cx1, cy1, cx2, cy2, cid,
           interpret=False):
    row_spec = pl.BlockSpec((1, 1, _R), lambda g: (g, 0, 0))
    col_spec = pl.BlockSpec((_NCB, _CB), lambda g: (0, 0))
    return pl.pallas_call(
        _match_body,
        grid=(_NP // _R,),
        in_specs=[row_spec] * 6 + [col_spec] * 5,
        out_specs=[row_spec, row_spec, row_spec],
        out_shape=[
            jax.ShapeDtypeStruct((_NCB, 1, _R), jnp.float32),
            jax.ShapeDtypeStruct((_NCB, 1, _R), jnp.float32),
            jax.ShapeDtypeStruct((_NCB, 1, _R), jnp.int32),
        ],
        compiler_params=pltpu.CompilerParams(
            dimension_semantics=("parallel",)),
        interpret=interpret,
    )(rx1, ry1, rx2, ry2, rid, rsc, cx1, cy1, cx2, cy2, cid)


def _make_sc_paint():
    info = plsc.get_sparse_core_info()
    nc, ns = info.num_cores, info.num_subcores
    nw = nc * ns
    rows = _HS // nw
    strip = rows * _HS
    mesh = plsc.VectorSubcoreMesh(core_axis_name="c", subcore_axis_name="s")

    @functools.partial(
        pl.kernel,
        mesh=mesh,
        out_type=jax.ShapeDtypeStruct((4 * _HS * _HS,), jnp.float32),
        compiler_params=pltpu.CompilerParams(needs_layout_passes=False),
        scratch_types=[
            pltpu.VMEM((_NBE,), jnp.int32),
            pltpu.VMEM((_NBE,), jnp.float32),
            pltpu.VMEM((_NPE,), jnp.float32),
            pltpu.VMEM((_NPE,), jnp.int32),
            pltpu.VMEM((_NP,), jnp.float32),
            pltpu.VMEM((_NPE,), jnp.float32),
            pltpu.VMEM((_NBE,), jnp.int32),
            pltpu.VMEM((16,), jnp.int32),
            pltpu.VMEM((16,), jnp.float32),
            pltpu.VMEM((strip,), jnp.float32),
            pltpu.VMEM((4 * strip,), jnp.float32),
        ],
    )
    def sc_paint(pack_h, dv_h, ign_h, bidx_h, sp_h,
                 out_h,
                 packv, valsv, ignv, bidxv, spv, igmv,
                 listv, kscr, vscr, stripv, upv):
        wid = lax.axis_index("s") * nc + lax.axis_index("c")
        y0 = wid * rows
        pltpu.sync_copy(pack_h, packv)
        pltpu.sync_copy(dv_h, valsv.at[pl.ds(0, _NP)])
        pltpu.sync_copy(ign_h, ignv)
        pltpu.sync_copy(bidx_h, bidxv)
        pltpu.sync_copy(sp_h, spv)

        iota = lax.iota(jnp.int32, 16)
        zero16 = jnp.zeros((16,), jnp.float32)
        neg16 = jnp.full((16,), -1.0, jnp.float32)

        def z_body(i, _):
            stripv[pl.ds(i * 16, 16)] = zero16
            return 0

        lax.fori_loop(0, strip // 16, z_body, 0)

        def n_body(i, _):
            igmv[pl.ds(i * 16, 16)] = neg16
            return 0

        lax.fori_loop(0, _NPE // 16, n_body, 0)

        def cand_body(g, off):
            b0 = g * 16
            pw = packv[pl.ds(b0, 16)]
            ylo = lax.shift_right_logical(pw, 8) & 255
            yhi = lax.shift_right_logical(pw, 24) & 255
            m = (ylo < y0 + rows) & (yhi > y0)
            plsc.store_compressed(listv.at[pl.ds(off, 16)], b0 + iota,
                                  mask=m)
            return off + plsc.all_reduce_population_count(m)[0]

        cnt = lax.fori_loop(0, _NB // 16, cand_body, 0)

        def igm_body(c, _):
            j16 = bidxv[pl.ds(c * 16, 16)]
            g16 = ignv[pl.ds(c * 16, 16)]
            key = jnp.where(g16 >= 0.0, j16, _NP)
            ks, vs = plsc.sort_key_val(key, g16)
            kscr[...] = ks
            for s in (1, 2, 4, 8):
                vscr[...] = vs
                idx = jnp.maximum(iota - s, 0)
                pk = plsc.load_gather(kscr, [idx])
                pv = plsc.load_gather(vscr, [idx])
                take = (iota >= s) & (pk == ks)
                vs = jnp.where(take, jnp.maximum(vs, pv), vs)
            nk = plsc.load_gather(kscr, [jnp.minimum(iota + 1, 15)])
            last = (iota == 15) | (nk != ks)
            msk = last & (ks < _NP)
            cur = plsc.load_gather(igmv, [ks])
            plsc.store_scatter(igmv, [ks], jnp.maximum(cur, vs), mask=msk)
            return 0

        lax.fori_loop(0, _NP // 16, igm_body, 0)

        def pv_body(i, _):
            g = igmv[pl.ds(i * 16, 16)]
            s = spv[pl.ds(i * 16, 16)]
            valsv[pl.ds(_NP + i * 16, 16)] = jnp.where(g >= 0.0, g * s, s)
            return 0

        lax.fori_loop(0, _NP // 16, pv_body, 0)

        def body(t, _):
            b = listv[pl.ds(t, 16)][0]
            pw = packv[pl.ds(b, 16)][0]
            x1 = pw & 255
            y1 = lax.shift_right_logical(pw, 8) & 255
            x2 = lax.shift_right_logical(pw, 16) & 255
            y2 = lax.shift_right_logical(pw, 24) & 255
            v = valsv[pl.ds(b, 16)][0]
            lo = jnp.maximum(y1, y0)
            hi = jnp.minimum(y2, y0 + rows)
            msk = (x1 + iota) < x2

            def row(y, _):
                idx = (y - y0) * _HS + x1 + iota
                cur = plsc.load_gather(stripv, [idx])
                plsc.store_scatter(stripv, [idx],
                                   jnp.maximum(cur, v), mask=msk)
                return 0

            lax.fori_loop(lo, hi, row, 0)
            return 0

        lax.fori_loop(0, cnt, body, 0)

        def up_body(r, _):
            def chunk(k, _):
                src = plsc.load_gather(
                    stripv,
                    [r * _HS + lax.shift_right_logical(k * 16 + iota, 1)])
                upv[pl.ds(r * 4 * _HS + k * 16, 16)] = src
                upv[pl.ds(r * 4 * _HS + 2 * _HS + k * 16, 16)] = src
                return 0

            lax.fori_loop(0, 2 * _HS // 16, chunk, 0)
            return 0

        lax.fori_loop(0, rows, up_body, 0)
        pltpu.sync_copy(upv, out_h.at[pl.ds(wid * 4 * strip, 4 * strip)])

    return sc_paint


def _pad1(x):
    return jnp.pad(x, (0, _NP - x.shape[0]))


def kernel(inputs, dets, dets_prev):
    n_, c_, h, w = inputs.shape
    q = jnp.floor(dets[:, :4] * 0.5)
    qp = jnp.floor(dets_prev[:, :4] * 0.5)

    rx1 = _pad1(q[:, 0]).reshape(_NCB, 1, _CB)
    ry1 = _pad1(q[:, 1]).reshape(_NCB, 1, _CB)
    rx2 = _pad1(q[:, 2]).reshape(_NCB, 1, _CB)
    ry2 = _pad1(q[:, 3]).reshape(_NCB, 1, _CB)
    rid = _pad1(dets[:, 5]).reshape(_NCB, 1, _CB)
    rsc = _pad1(dets[:, 4]).reshape(_NCB, 1, _CB)
    cx1 = _pad1(qp[:, 0]).reshape(_NCB, _CB)
    cy1 = _pad1(qp[:, 1]).reshape(_NCB, _CB)
    cx2 = _pad1(qp[:, 2]).reshape(_NCB, _CB)
    cy2 = _pad1(qp[:, 3]).reshape(_NCB, _CB)
    cid = _pad1(dets_prev[:, 5]).reshape(_NCB, _CB)

    dv, ign, bidx = _match(rx1, ry1, rx2, ry2, rid, rsc,
                           cx1, cy1, cx2, cy2, cid)
    dv = dv.reshape(_NP)
    ign = ign.reshape(_NP)
    bidx = bidx.reshape(_NP)

    qi = q.astype(jnp.int32)
    qpi = qp.astype(jnp.int32)

    def packed(qb):
        return (qb[:, 0] | (qb[:, 1] << 8) | (qb[:, 2] << 16)
                | (qb[:, 3] << 24))

    pack = jnp.pad(jnp.concatenate([_pad1(packed(qi)), _pad1(packed(qpi))]),
                   (0, 16))
    sp = _pad1(dets_prev[:, 4])
    ign_e = jnp.pad(ign, (0, 16))
    bidx_e = jnp.pad(bidx, (0, 16))

    m_flat = _make_sc_paint()(pack, dv, ign_e, bidx_e, sp)
    return m_flat.reshape(1, 1, h, w)
